# Initial kernel scaffold; baseline (speedup 1.0000x reference)
import jax
import jax.numpy as jnp
from jax.experimental import pallas as pl


def kernel(x, edge_index, cluster, w1_0, b1_0, g_0, beta_0, w2_0, b2_0, w1_1, b1_1, g_1, beta_1, w2_1, b2_1, w1_2, b1_2, g_2, beta_2, w2_2, b2_2):
    raise NotImplementedError("write your pallas kernel here")


# speedup vs baseline: 1.5997x; 1.5997x over previous
import functools

import jax
import jax.numpy as jnp
from jax import lax
from jax.experimental import pallas as pl
from jax.experimental.pallas import tpu as pltpu
from jax.experimental.pallas import tpu_sc as plsc

N = 10000
NPAD = 10240
E = 320000
P = 512
H = 64
NW = 32
G = 128
NEG = float("-inf")

EPAD = E + 256
EPOOL = N + 240


def _mlp_pallas(xs, w1s, b1, g, beta, w2, b2):
    R = 320
    grid = NPAD // R
    nin = len(xs)
    dout = w2.shape[1]

    def body(*refs):
        xr = refs[:nin]
        w1r = refs[nin:2 * nin]
        b1r, gr, br, w2r, b2r, outr = refs[2 * nin:]
        h = jnp.dot(xr[0][...], w1r[0][...], preferred_element_type=jnp.float32)
        for i in range(1, nin):
            h = h + jnp.dot(xr[i][...], w1r[i][...],
                            preferred_element_type=jnp.float32)
        h = h + b1r[...]
        mu = jnp.mean(h, axis=-1, keepdims=True)
        var = jnp.mean((h - mu) ** 2, axis=-1, keepdims=True)
        h = (h - mu) / jnp.sqrt(var + 1e-5) * gr[...] + br[...]
        h = jnp.maximum(h, 0.0)
        outr[...] = jnp.dot(h, w2r[...],
                            preferred_element_type=jnp.float32) + b2r[...]

    in_specs = []
    for xi in xs:
        in_specs.append(pl.BlockSpec((R, xi.shape[1]), lambda i: (i, 0)))
    for wi in w1s:
        in_specs.append(pl.BlockSpec(wi.shape, lambda i: (0, 0)))
    in_specs.append(pl.BlockSpec((1, H), lambda i: (0, 0)))
    in_specs.append(pl.BlockSpec((1, H), lambda i: (0, 0)))
    in_specs.append(pl.BlockSpec((1, H), lambda i: (0, 0)))
    in_specs.append(pl.BlockSpec((H, dout), lambda i: (0, 0)))
    in_specs.append(pl.BlockSpec((1, dout), lambda i: (0, 0)))

    return pl.pallas_call(
        body,
        grid=(grid,),
        in_specs=in_specs,
        out_specs=pl.BlockSpec((R, dout), lambda i: (i, 0)),
        out_shape=jax.ShapeDtypeStruct((NPAD, dout), jnp.float32),
    )(*xs, *w1s, b1.reshape(1, H), g.reshape(1, H), beta.reshape(1, H),
      w2, b2.reshape(1, dout))


def _make_segmax(d, cseg, cpw, out_rows, cb_len):
    nj = d // 16
    mesh = plsc.VectorSubcoreMesh(core_axis_name="c", subcore_axis_name="s")

    @functools.partial(
        pl.kernel,
        out_type=jax.ShapeDtypeStruct((out_rows, d), jnp.float32),
        mesh=mesh,
        scratch_types=[
            pltpu.VMEM((cb_len,), jnp.int32),
            pltpu.VMEM((G,), jnp.int32),
            pltpu.VMEM((G,), jnp.int32),
            pltpu.VMEM((G, d), jnp.float32),
            pltpu.VMEM((cseg + 1, d), jnp.float32),
            pltpu.SemaphoreType.DMA,
        ],
    )
    def k(table, srcs, dsts, cb, out, cb_v, idx_v, dst_v, rows_v, acc_v, sem):
        w = lax.axis_index("s") * 2 + lax.axis_index("c")
        pltpu.sync_copy(cb, cb_v)

        def chunk_body(c, _):
            chunk = w * cpw + c
            m0 = chunk * cseg
            cvec = cb_v[pl.ds(chunk, 16)]
            e_lo = cvec[0]
            e_hi = cvec[1]

            def init_row(r, _):
                for j in range(nj):
                    acc_v[r, pl.ds(j * 16, 16)] = jnp.full((16,), NEG,
                                                           jnp.float32)
                return None
            lax.fori_loop(0, cseg, init_row, None)

            e0a = (e_lo // 8) * 8
            nb = (e_hi - e0a + G - 1) // G

            def batch(b, _):
                e0 = pl.multiple_of(e0a + b * G, 8)
                pltpu.sync_copy(srcs.at[pl.ds(e0, G)], idx_v)
                pltpu.sync_copy(dsts.at[pl.ds(e0, G)], dst_v)
                pltpu.async_copy(table.at[idx_v], rows_v, sem).wait()

                def group(gg, _):
                    dvec = dst_v[pl.ds(gg * 16, 16)] - m0
                    for lane in range(16):
                        dl = dvec[lane]
                        ok = jnp.logical_and(dl >= 0, dl < cseg)
                        dlc = jnp.where(ok, dl, cseg)
                        kk = gg * 16 + lane
                        for j in range(nj):
                            sl = pl.ds(j * 16, 16)
                            acc_v[dlc, sl] = jnp.maximum(acc_v[dlc, sl],
                                                         rows_v[kk, sl])
                    return None
                lax.fori_loop(0, G // 16, group, None)
                return None
            lax.fori_loop(0, nb, batch, None)

            def fix_row(r, _):
                for j in range(nj):
                    sl = pl.ds(j * 16, 16)
                    v = acc_v[r, sl]
                    acc_v[r, sl] = jnp.where(v == NEG, 0.0, v)
                return None
            lax.fori_loop(0, cseg, fix_row, None)
            pltpu.sync_copy(acc_v.at[pl.ds(0, cseg)], out.at[pl.ds(m0, cseg)])
            return None

        lax.fori_loop(0, cpw, chunk_body, None)

    return k


def _norm_pallas(pooled):
    def body(pr, outr):
        v = pr[...]
        ss = jnp.sum(v * v, axis=0, keepdims=True)
        nrm = jnp.sqrt(ss)
        nrm = jnp.where(nrm == 0.0, 1.0, nrm)
        outr[...] = v / nrm

    return pl.pallas_call(
        body,
        out_shape=jax.ShapeDtypeStruct(pooled.shape, pooled.dtype),
    )(pooled)


def kernel(x, edge_index, cluster,
           w1_0, b1_0, g_0, beta_0, w2_0, b2_0,
           w1_1, b1_1, g_1, beta_1, w2_1, b2_1,
           w1_2, b1_2, g_2, beta_2, w2_2, b2_2):
    src = edge_index[0]
    dst = edge_index[1]

    dst_s, src_s = lax.sort_key_val(dst, src)
    src_p = jnp.concatenate(
        [src_s, jnp.zeros((EPAD - E,), jnp.int32)])
    dst_p = jnp.concatenate(
        [dst_s, jnp.full((EPAD - E,), NPAD, jnp.int32)])
    cb = jnp.searchsorted(
        dst_s, jnp.arange(0, NPAD + 1, 64, dtype=jnp.int32)).astype(jnp.int32)
    cb = jnp.concatenate([cb, jnp.full((15,), E, jnp.int32)])

    rid_p = jnp.concatenate(
        [jnp.arange(N, dtype=jnp.int32), jnp.zeros((EPOOL - N,), jnp.int32)])
    clu_p = jnp.concatenate(
        [cluster, jnp.full((EPOOL - N,), P, jnp.int32)])
    cbp = jnp.searchsorted(
        cluster, jnp.arange(0, P + 1, 16, dtype=jnp.int32)).astype(jnp.int32)
    cbp = jnp.concatenate([cbp, jnp.full((15,), N, jnp.int32)])

    x_p = jnp.pad(x, ((0, NPAD - N), (0, 0)))

    segmax_128 = _make_segmax(128, 64, 5, NPAD, 176)
    segmax_256 = _make_segmax(256, 64, 5, NPAD, 176)
    segmax_512 = _make_segmax(512, 64, 5, NPAD, 176)
    segmax_pool = _make_segmax(512, 16, 1, P, 48)

    xt0 = _mlp_pallas([x_p], [w1_0], b1_0, g_0, beta_0, w2_0, b2_0)
    agg0 = segmax_128(xt0, src_p, dst_p, cb)
    xt1 = _mlp_pallas([xt0, agg0], [w1_1[:128], w1_1[128:]],
                      b1_1, g_1, beta_1, w2_1, b2_1)
    agg1 = segmax_256(xt1, src_p, dst_p, cb)
    xt2 = _mlp_pallas([xt1, agg1], [w1_2[:256], w1_2[256:]],
                      b1_2, g_2, beta_2, w2_2, b2_2)
    agg2 = segmax_512(xt2, src_p, dst_p, cb)

    pool_a = segmax_pool(xt2, rid_p, clu_p, cbp)
    pool_b = segmax_pool(agg2, rid_p, clu_p, cbp)
    pooled = jnp.concatenate([pool_a, pool_b], axis=1)

    return _norm_pallas(pooled)


# speedup vs baseline: 3.8386x; 2.3995x over previous
import functools

import jax
import jax.numpy as jnp
from jax import lax
from jax.experimental import pallas as pl
from jax.experimental.pallas import tpu as pltpu
from jax.experimental.pallas import tpu_sc as plsc

N = 10000
NPAD = 10240
E = 320000
P = 512
H = 64
NW = 32
G = 128
NEG = float("-inf")

EPAD = E + 256
EPOOL = N + 240


def _mlp_pallas(xs, w1s, b1, g, beta, w2, b2):
    R = 320
    grid = NPAD // R
    nin = len(xs)
    dout = w2.shape[1]

    def body(*refs):
        xr = refs[:nin]
        w1r = refs[nin:2 * nin]
        b1r, gr, br, w2r, b2r, outr = refs[2 * nin:]
        h = jnp.dot(xr[0][...], w1r[0][...], preferred_element_type=jnp.float32)
        for i in range(1, nin):
            h = h + jnp.dot(xr[i][...], w1r[i][...],
                            preferred_element_type=jnp.float32)
        h = h + b1r[...]
        mu = jnp.mean(h, axis=-1, keepdims=True)
        var = jnp.mean((h - mu) ** 2, axis=-1, keepdims=True)
        h = (h - mu) / jnp.sqrt(var + 1e-5) * gr[...] + br[...]
        h = jnp.maximum(h, 0.0)
        outr[...] = jnp.dot(h, w2r[...],
                            preferred_element_type=jnp.float32) + b2r[...]

    in_specs = []
    for xi in xs:
        in_specs.append(pl.BlockSpec((R, xi.shape[1]), lambda i: (i, 0)))
    for wi in w1s:
        in_specs.append(pl.BlockSpec(wi.shape, lambda i: (0, 0)))
    in_specs.append(pl.BlockSpec((1, H), lambda i: (0, 0)))
    in_specs.append(pl.BlockSpec((1, H), lambda i: (0, 0)))
    in_specs.append(pl.BlockSpec((1, H), lambda i: (0, 0)))
    in_specs.append(pl.BlockSpec((H, dout), lambda i: (0, 0)))
    in_specs.append(pl.BlockSpec((1, dout), lambda i: (0, 0)))

    return pl.pallas_call(
        body,
        grid=(grid,),
        in_specs=in_specs,
        out_specs=pl.BlockSpec((R, dout), lambda i: (i, 0)),
        out_shape=jax.ShapeDtypeStruct((NPAD, dout), jnp.float32),
    )(*xs, *w1s, b1.reshape(1, H), g.reshape(1, H), beta.reshape(1, H),
      w2, b2.reshape(1, dout))


def _make_segmax(d, cseg, cpw, out_rows, cb_len, g, epad):
    nj = d // 16
    mesh = plsc.VectorSubcoreMesh(core_axis_name="c", subcore_axis_name="s")

    @functools.partial(
        pl.kernel,
        out_type=jax.ShapeDtypeStruct((out_rows * d,), jnp.float32),
        mesh=mesh,
        scratch_types=[
            pltpu.VMEM((cb_len,), jnp.int32),
            pltpu.VMEM((g,), jnp.int32),
            pltpu.VMEM((g,), jnp.int32),
            pltpu.VMEM((g,), jnp.int32),
            pltpu.VMEM((g,), jnp.int32),
            pltpu.VMEM((g, d), jnp.float32),
            pltpu.VMEM((g, d), jnp.float32),
            pltpu.VMEM(((cseg + 1) * d,), jnp.float32),
            pltpu.SemaphoreType.DMA,
            pltpu.SemaphoreType.DMA,
        ],
    )
    def k(table, srcs, dsts, cb, out,
          cb_v, dva, dvb, ixa, ixb, rwa, rwb, acc_v, sma, smb):
        w = lax.axis_index("s") * 2 + lax.axis_index("c")
        pltpu.sync_copy(cb, cb_v)
        zero = jnp.zeros((16,), jnp.float32)
        neg = jnp.full((16,), NEG, jnp.float32)

        def clamp(e):
            return pl.multiple_of(jnp.minimum(e, epad - g), 8)

        def chunk_body(c, _):
            chunk = w * cpw + c
            m0 = chunk * cseg
            cvec = cb_v[pl.ds(chunk, 16)]
            e_lo = cvec[0]
            e_hi = cvec[1]

            def process(dv_ref, rw_ref, carry):
                def step(t, carry):
                    dvec = dv_ref[pl.ds(t * 16, 16)] - m0
                    okv = jnp.logical_and(dvec >= 0, dvec < cseg)
                    dlcv = jnp.where(okv, dvec, cseg)
                    for u in range(16):
                        kk = t * 16 + u
                        prev = carry[0]
                        accs = list(carry[1:])
                        dlc = dlcv[u]
                        change = dlc != prev

                        @pl.when(change)
                        def _():
                            for j in range(nj):
                                acc_v[pl.ds(prev * d + j * 16, 16)] = accs[j]

                        carry = (dlc,) + tuple(
                            jnp.maximum(jnp.where(change, neg, accs[j]),
                                        rw_ref[kk, pl.ds(j * 16, 16)])
                            for j in range(nj))
                    return carry
                return lax.fori_loop(0, g // 16, step, carry)

            def init_row(r, _):
                for j in range(nj):
                    acc_v[pl.ds(r * d + j * 16, 16)] = zero
                return None
            lax.fori_loop(0, cseg, init_row, None)

            e0a = (e_lo // 8) * 8
            npair = jnp.maximum((e_hi - e0a + 2 * g - 1) // (2 * g), 1)

            e0 = clamp(e0a)
            pltpu.sync_copy(srcs.at[pl.ds(e0, g)], ixa)
            pltpu.sync_copy(dsts.at[pl.ds(e0, g)], dva)
            pltpu.async_copy(table.at[ixa], rwa, sma)

            carry0 = (jnp.int32(cseg),) + tuple(neg for _ in range(nj))

            def pair(i, carry):
                e1 = clamp(e0a + (2 * i + 1) * g)
                pltpu.sync_copy(srcs.at[pl.ds(e1, g)], ixb)
                pltpu.sync_copy(dsts.at[pl.ds(e1, g)], dvb)
                pltpu.async_copy(table.at[ixb], rwb, smb)
                pltpu.make_async_copy(table.at[ixa], rwa, sma).wait()
                carry = process(dva, rwa, carry)
                e2 = clamp(e0a + (2 * i + 2) * g)
                pltpu.sync_copy(srcs.at[pl.ds(e2, g)], ixa)
                pltpu.sync_copy(dsts.at[pl.ds(e2, g)], dva)
                pltpu.async_copy(table.at[ixa], rwa, sma)
                pltpu.make_async_copy(table.at[ixb], rwb, smb).wait()
                carry = process(dvb, rwb, carry)
                return carry

            carry = lax.fori_loop(0, npair, pair, carry0)

            pltpu.make_async_copy(table.at[ixa], rwa, sma).wait()
            prev = carry[0]
            for j in range(nj):
                acc_v[pl.ds(prev * d + j * 16, 16)] = carry[1 + j]

            pltpu.sync_copy(acc_v.at[pl.ds(0, cseg * d)],
                            out.at[pl.ds(m0 * d, cseg * d)])
            return None

        lax.fori_loop(0, cpw, chunk_body, None)

    return k


def _norm_pallas(pooled):
    def body(pr, outr):
        v = pr[...]
        ss = jnp.sum(v * v, axis=0, keepdims=True)
        nrm = jnp.sqrt(ss)
        nrm = jnp.where(nrm == 0.0, 1.0, nrm)
        outr[...] = v / nrm

    return pl.pallas_call(
        body,
        out_shape=jax.ShapeDtypeStruct(pooled.shape, pooled.dtype),
    )(pooled)


def kernel(x, edge_index, cluster,
           w1_0, b1_0, g_0, beta_0, w2_0, b2_0,
           w1_1, b1_1, g_1, beta_1, w2_1, b2_1,
           w1_2, b1_2, g_2, beta_2, w2_2, b2_2):
    src = edge_index[0]
    dst = edge_index[1]

    dst_s, src_s = lax.sort_key_val(dst, src)
    src_p = jnp.concatenate(
        [src_s, jnp.zeros((EPAD - E,), jnp.int32)])
    dst_p = jnp.concatenate(
        [dst_s, jnp.full((EPAD - E,), NPAD, jnp.int32)])
    cb = jnp.searchsorted(
        dst_s, jnp.arange(0, NPAD + 1, 64, dtype=jnp.int32)).astype(jnp.int32)
    cb = jnp.concatenate([cb, jnp.full((15,), E, jnp.int32)])

    rid_p = jnp.concatenate(
        [jnp.arange(N, dtype=jnp.int32), jnp.zeros((EPOOL - N,), jnp.int32)])
    clu_p = jnp.concatenate(
        [cluster, jnp.full((EPOOL - N,), P, jnp.int32)])
    cbp = jnp.searchsorted(
        cluster, jnp.arange(0, P + 1, 16, dtype=jnp.int32)).astype(jnp.int32)
    cbp = jnp.concatenate([cbp, jnp.full((15,), N, jnp.int32)])

    x_p = jnp.pad(x, ((0, NPAD - N), (0, 0)))

    segmax_128 = _make_segmax(128, 64, 5, NPAD, 176, 128, EPAD)
    segmax_256 = _make_segmax(256, 64, 5, NPAD, 176, 128, EPAD)
    segmax_512 = _make_segmax(512, 64, 5, NPAD, 176, 64, EPAD)
    segmax_pool = _make_segmax(512, 16, 1, P, 48, 64, EPOOL)

    xt0 = _mlp_pallas([x_p], [w1_0], b1_0, g_0, beta_0, w2_0, b2_0)
    agg0 = segmax_128(xt0, src_p, dst_p, cb).reshape(NPAD, 128)
    xt1 = _mlp_pallas([xt0, agg0], [w1_1[:128], w1_1[128:]],
                      b1_1, g_1, beta_1, w2_1, b2_1)
    agg1 = segmax_256(xt1, src_p, dst_p, cb).reshape(NPAD, 256)
    xt2 = _mlp_pallas([xt1, agg1], [w1_2[:256], w1_2[256:]],
                      b1_2, g_2, beta_2, w2_2, b2_2)
    agg2 = segmax_512(xt2, src_p, dst_p, cb).reshape(NPAD, 512)

    pool_a = segmax_pool(xt2, rid_p, clu_p, cbp).reshape(P, 512)
    pool_b = segmax_pool(agg2, rid_p, clu_p, cbp).reshape(P, 512)
    pooled = jnp.concatenate([pool_a, pool_b], axis=1)

    return _norm_pallas(pooled)


# speedup vs baseline: 4.4245x; 1.1526x over previous
import functools

import jax
import jax.numpy as jnp
from jax import lax
from jax.experimental import pallas as pl
from jax.experimental.pallas import tpu as pltpu
from jax.experimental.pallas import tpu_sc as plsc

N = 10000
NPAD = 10240
E = 320000
P = 512
H = 64
NW = 32
G = 128
NEG = float("-inf")

SB = 2048
SBP = 512
EPAD = E + 2 * SB + 128
EPOOL = N + 2 * SBP + 128


def _mlp_pallas(xs, w1s, b1, g, beta, w2, b2):
    R = 320
    grid = NPAD // R
    nin = len(xs)
    dout = w2.shape[1]

    def body(*refs):
        xr = refs[:nin]
        w1r = refs[nin:2 * nin]
        b1r, gr, br, w2r, b2r, outr = refs[2 * nin:]
        h = jnp.dot(xr[0][...], w1r[0][...], preferred_element_type=jnp.float32)
        for i in range(1, nin):
            h = h + jnp.dot(xr[i][...], w1r[i][...],
                            preferred_element_type=jnp.float32)
        h = h + b1r[...]
        mu = jnp.mean(h, axis=-1, keepdims=True)
        var = jnp.mean((h - mu) ** 2, axis=-1, keepdims=True)
        h = (h - mu) / jnp.sqrt(var + 1e-5) * gr[...] + br[...]
        h = jnp.maximum(h, 0.0)
        outr[...] = jnp.dot(h, w2r[...],
                            preferred_element_type=jnp.float32) + b2r[...]

    in_specs = []
    for xi in xs:
        in_specs.append(pl.BlockSpec((R, xi.shape[1]), lambda i: (i, 0)))
    for wi in w1s:
        in_specs.append(pl.BlockSpec(wi.shape, lambda i: (0, 0)))
    in_specs.append(pl.BlockSpec((1, H), lambda i: (0, 0)))
    in_specs.append(pl.BlockSpec((1, H), lambda i: (0, 0)))
    in_specs.append(pl.BlockSpec((1, H), lambda i: (0, 0)))
    in_specs.append(pl.BlockSpec((H, dout), lambda i: (0, 0)))
    in_specs.append(pl.BlockSpec((1, dout), lambda i: (0, 0)))

    return pl.pallas_call(
        body,
        grid=(grid,),
        in_specs=in_specs,
        out_specs=pl.BlockSpec((R, dout), lambda i: (i, 0)),
        out_shape=jax.ShapeDtypeStruct((NPAD, dout), jnp.float32),
    )(*xs, *w1s, b1.reshape(1, H), g.reshape(1, H), beta.reshape(1, H),
      w2, b2.reshape(1, dout))


def _make_segmax(d, cseg, cpw, out_rows, cb_len, g, sb_sz, epad):
    nj = d // 16
    mesh = plsc.VectorSubcoreMesh(core_axis_name="c", subcore_axis_name="s")

    @functools.partial(
        pl.kernel,
        out_type=jax.ShapeDtypeStruct((out_rows * d,), jnp.float32),
        mesh=mesh,
        scratch_types=[
            pltpu.VMEM((cb_len,), jnp.int32),
            pltpu.VMEM((sb_sz + g,), jnp.int32),
            pltpu.VMEM((sb_sz + g,), jnp.int32),
            pltpu.VMEM((g, d), jnp.float32),
            pltpu.VMEM((g, d), jnp.float32),
            pltpu.VMEM(((cseg + 1) * d,), jnp.float32),
            pltpu.SemaphoreType.DMA,
            pltpu.SemaphoreType.DMA,
        ],
    )
    def k(table, srcs, dsts, cb, out,
          cb_v, ixs, dvs, rwa, rwb, acc_v, sma, smb):
        w = lax.axis_index("s") * 2 + lax.axis_index("c")
        pltpu.sync_copy(cb, cb_v)
        zero = jnp.zeros((16,), jnp.float32)
        neg = jnp.full((16,), NEG, jnp.float32)

        def chunk_body(c, _):
            chunk = w * cpw + c
            m0 = chunk * cseg
            cvec = cb_v[pl.ds(chunk, 16)]
            e_lo = cvec[0]
            e_hi = cvec[1]

            def process(off, rw_ref, carry):
                def step(t, carry):
                    dvec = dvs[pl.ds(off + t * 16, 16)] - m0
                    okv = jnp.logical_and(dvec >= 0, dvec < cseg)
                    dlcv = jnp.where(okv, dvec, cseg)
                    for u in range(16):
                        kk = t * 16 + u
                        prev = carry[0]
                        accs = list(carry[1:])
                        dlc = dlcv[u]
                        change = dlc != prev

                        @pl.when(change)
                        def _():
                            for j in range(nj):
                                acc_v[pl.ds(prev * d + j * 16, 16)] = accs[j]

                        carry = (dlc,) + tuple(
                            jnp.maximum(jnp.where(change, neg, accs[j]),
                                        rw_ref[kk, pl.ds(j * 16, 16)])
                            for j in range(nj))
                    return carry
                return lax.fori_loop(0, g // 16, step, carry)

            def init_row(r, _):
                for j in range(nj):
                    acc_v[pl.ds(r * d + j * 16, 16)] = zero
                return None
            lax.fori_loop(0, cseg, init_row, None)

            e0a = (e_lo // 8) * 8
            nsb = (e_hi - e0a + sb_sz - 1) // sb_sz

            def super_body(sb, carry):
                est = pl.multiple_of(e0a + sb * sb_sz, 8)
                pltpu.sync_copy(srcs.at[pl.ds(est, sb_sz + g)], ixs)
                pltpu.sync_copy(dsts.at[pl.ds(est, sb_sz + g)], dvs)
                rem = jnp.minimum(e_hi - est, sb_sz)
                npi = (rem + 2 * g - 1) // (2 * g)
                pltpu.async_copy(table.at[ixs.at[pl.ds(0, g)]], rwa, sma)

                def pair(i, carry):
                    pltpu.async_copy(
                        table.at[ixs.at[pl.ds((2 * i + 1) * g, g)]], rwb, smb)
                    pltpu.make_async_copy(table.at[ixs.at[pl.ds(0, g)]], rwa,
                                          sma).wait()
                    carry = process(2 * i * g, rwa, carry)
                    pltpu.async_copy(
                        table.at[ixs.at[pl.ds((2 * i + 2) * g, g)]], rwa, sma)
                    pltpu.make_async_copy(table.at[ixs.at[pl.ds(0, g)]], rwb,
                                          smb).wait()
                    carry = process((2 * i + 1) * g, rwb, carry)
                    return carry

                carry = lax.fori_loop(0, npi, pair, carry)
                pltpu.make_async_copy(table.at[ixs.at[pl.ds(0, g)]], rwa,
                                          sma).wait()
                return carry

            carry0 = (jnp.int32(cseg),) + tuple(neg for _ in range(nj))
            carry = lax.fori_loop(0, nsb, super_body, carry0)

            prev = carry[0]
            for j in range(nj):
                acc_v[pl.ds(prev * d + j * 16, 16)] = carry[1 + j]

            pltpu.sync_copy(acc_v.at[pl.ds(0, cseg * d)],
                            out.at[pl.ds(m0 * d, cseg * d)])
            return None

        lax.fori_loop(0, cpw, chunk_body, None)

    return k


def _norm_pallas(pooled):
    def body(pr, outr):
        v = pr[...]
        ss = jnp.sum(v * v, axis=0, keepdims=True)
        nrm = jnp.sqrt(ss)
        nrm = jnp.where(nrm == 0.0, 1.0, nrm)
        outr[...] = v / nrm

    return pl.pallas_call(
        body,
        out_shape=jax.ShapeDtypeStruct(pooled.shape, pooled.dtype),
    )(pooled)


def kernel(x, edge_index, cluster,
           w1_0, b1_0, g_0, beta_0, w2_0, b2_0,
           w1_1, b1_1, g_1, beta_1, w2_1, b2_1,
           w1_2, b1_2, g_2, beta_2, w2_2, b2_2):
    src = edge_index[0]
    dst = edge_index[1]

    dst_s, src_s = lax.sort_key_val(dst, src)
    src_p = jnp.concatenate(
        [src_s, jnp.zeros((EPAD - E,), jnp.int32)])
    dst_p = jnp.concatenate(
        [dst_s, jnp.full((EPAD - E,), NPAD, jnp.int32)])
    cb = jnp.searchsorted(
        dst_s, jnp.arange(0, NPAD + 1, 64, dtype=jnp.int32)).astype(jnp.int32)
    cb = jnp.concatenate([cb, jnp.full((15,), E, jnp.int32)])

    rid_p = jnp.concatenate(
        [jnp.arange(N, dtype=jnp.int32), jnp.zeros((EPOOL - N,), jnp.int32)])
    clu_p = jnp.concatenate(
        [cluster, jnp.full((EPOOL - N,), P, jnp.int32)])
    cbp = jnp.searchsorted(
        cluster, jnp.arange(0, P + 1, 16, dtype=jnp.int32)).astype(jnp.int32)
    cbp = jnp.concatenate([cbp, jnp.full((15,), N, jnp.int32)])

    x_p = jnp.pad(x, ((0, NPAD - N), (0, 0)))

    segmax_128 = _make_segmax(128, 64, 5, NPAD, 176, 128, SB, EPAD)
    segmax_256 = _make_segmax(256, 64, 5, NPAD, 176, 128, SB, EPAD)
    segmax_512 = _make_segmax(512, 64, 5, NPAD, 176, 64, SB, EPAD)
    segmax_pool = _make_segmax(512, 16, 1, P, 48, 64, SBP, EPOOL)

    xt0 = _mlp_pallas([x_p], [w1_0], b1_0, g_0, beta_0, w2_0, b2_0)
    agg0 = segmax_128(xt0, src_p, dst_p, cb).reshape(NPAD, 128)
    xt1 = _mlp_pallas([xt0, agg0], [w1_1[:128], w1_1[128:]],
                      b1_1, g_1, beta_1, w2_1, b2_1)
    agg1 = segmax_256(xt1, src_p, dst_p, cb).reshape(NPAD, 256)
    xt2 = _mlp_pallas([xt1, agg1], [w1_2[:256], w1_2[256:]],
                      b1_2, g_2, beta_2, w2_2, b2_2)
    agg2 = segmax_512(xt2, src_p, dst_p, cb).reshape(NPAD, 512)

    pool_a = segmax_pool(xt2, rid_p, clu_p, cbp).reshape(P, 512)
    pool_b = segmax_pool(agg2, rid_p, clu_p, cbp).reshape(P, 512)
    pooled = jnp.concatenate([pool_a, pool_b], axis=1)

    return _norm_pallas(pooled)


# speedup vs baseline: 4.4497x; 1.0057x over previous
import functools

import jax
import jax.numpy as jnp
from jax import lax
from jax.experimental import pallas as pl
from jax.experimental.pallas import tpu as pltpu
from jax.experimental.pallas import tpu_sc as plsc

N = 10000
NPAD = 10240
E = 320000
P = 512
H = 64
NW = 32
G = 128
NEG = float("-inf")

SB = 2048
SBP = 512
EPAD = E + 2 * SB + 128
EPOOL = N + 2 * SBP + 128


def _mlp_pallas(xs, w1s, b1, g, beta, w2, b2):
    R = 320
    grid = NPAD // R
    nin = len(xs)
    dout = w2.shape[1]

    def body(*refs):
        xr = refs[:nin]
        w1r = refs[nin:2 * nin]
        b1r, gr, br, w2r, b2r, outr = refs[2 * nin:]
        h = jnp.dot(xr[0][...], w1r[0][...], preferred_element_type=jnp.float32)
        for i in range(1, nin):
            h = h + jnp.dot(xr[i][...], w1r[i][...],
                            preferred_element_type=jnp.float32)
        h = h + b1r[...]
        mu = jnp.mean(h, axis=-1, keepdims=True)
        var = jnp.mean((h - mu) ** 2, axis=-1, keepdims=True)
        h = (h - mu) / jnp.sqrt(var + 1e-5) * gr[...] + br[...]
        h = jnp.maximum(h, 0.0)
        outr[...] = jnp.dot(h, w2r[...],
                            preferred_element_type=jnp.float32) + b2r[...]

    in_specs = []
    for xi in xs:
        in_specs.append(pl.BlockSpec((R, xi.shape[1]), lambda i: (i, 0)))
    for wi in w1s:
        in_specs.append(pl.BlockSpec(wi.shape, lambda i: (0, 0)))
    in_specs.append(pl.BlockSpec((1, H), lambda i: (0, 0)))
    in_specs.append(pl.BlockSpec((1, H), lambda i: (0, 0)))
    in_specs.append(pl.BlockSpec((1, H), lambda i: (0, 0)))
    in_specs.append(pl.BlockSpec((H, dout), lambda i: (0, 0)))
    in_specs.append(pl.BlockSpec((1, dout), lambda i: (0, 0)))

    return pl.pallas_call(
        body,
        grid=(grid,),
        in_specs=in_specs,
        out_specs=pl.BlockSpec((R, dout), lambda i: (i, 0)),
        out_shape=jax.ShapeDtypeStruct((NPAD, dout), jnp.float32),
    )(*xs, *w1s, b1.reshape(1, H), g.reshape(1, H), beta.reshape(1, H),
      w2, b2.reshape(1, dout))


def _make_segmax(d, cseg, cpw, out_rows, cb_len, g, sb_sz, epad):
    nj = d // 16
    mesh = plsc.VectorSubcoreMesh(core_axis_name="c", subcore_axis_name="s")

    @functools.partial(
        pl.kernel,
        out_type=jax.ShapeDtypeStruct((out_rows * d,), jnp.float32),
        mesh=mesh,
        scratch_types=[
            pltpu.VMEM((cb_len,), jnp.int32),
            pltpu.VMEM((sb_sz + g,), jnp.int32),
            pltpu.VMEM((sb_sz + g,), jnp.int32),
            pltpu.VMEM((g, d), jnp.float32),
            pltpu.VMEM((g, d), jnp.float32),
            pltpu.VMEM(((cseg + 1) * d,), jnp.float32),
            pltpu.SemaphoreType.DMA,
            pltpu.SemaphoreType.DMA,
        ],
    )
    def k(table, srcs, dsts, cb, out,
          cb_v, ixs, dvs, rwa, rwb, acc_v, sma, smb):
        w = lax.axis_index("s") * 2 + lax.axis_index("c")
        pltpu.sync_copy(cb, cb_v)
        zero = jnp.zeros((16,), jnp.float32)
        neg = jnp.full((16,), NEG, jnp.float32)

        def chunk_body(c, _):
            chunk = w * cpw + c
            m0 = chunk * cseg
            cvec = cb_v[pl.ds(chunk, 16)]
            e_lo = cvec[0]
            e_hi = cvec[1]

            def process(off, rw_ref, carry):
                def step(t, carry):
                    dvec = dvs[pl.ds(off + t * 16, 16)] - m0
                    okv = jnp.logical_and(dvec >= 0, dvec < cseg)
                    dlcv = jnp.where(okv, dvec, cseg)
                    for u in range(16):
                        kk = t * 16 + u
                        prev = carry[0]
                        accs = list(carry[1:])
                        dlc = dlcv[u]
                        change = dlc != prev

                        @pl.when(change)
                        def _():
                            for j in range(nj):
                                acc_v[pl.ds(prev * d + j * 16, 16)] = accs[j]

                        carry = (dlc,) + tuple(
                            jnp.maximum(jnp.where(change, neg, accs[j]),
                                        rw_ref[kk, pl.ds(j * 16, 16)])
                            for j in range(nj))
                    return carry
                return lax.fori_loop(0, g // 16, step, carry)

            def init_row(r, _):
                for j in range(nj):
                    acc_v[pl.ds(r * d + j * 16, 16)] = zero
                return None
            lax.fori_loop(0, cseg, init_row, None)

            e0a = (e_lo // 8) * 8
            nsb = (e_hi - e0a + sb_sz - 1) // sb_sz

            def super_body(sb, carry):
                est = pl.multiple_of(e0a + sb * sb_sz, 8)
                pltpu.sync_copy(srcs.at[pl.ds(est, sb_sz + g)], ixs)
                pltpu.sync_copy(dsts.at[pl.ds(est, sb_sz + g)], dvs)
                rem = jnp.minimum(e_hi - est, sb_sz)
                npi = (rem + 2 * g - 1) // (2 * g)
                pltpu.async_copy(table.at[ixs.at[pl.ds(0, g)]], rwa, sma)

                def pair(i, carry):
                    pltpu.async_copy(
                        table.at[ixs.at[pl.ds((2 * i + 1) * g, g)]], rwb, smb)
                    pltpu.make_async_copy(table.at[ixs.at[pl.ds(0, g)]], rwa,
                                          sma).wait()
                    carry = process(2 * i * g, rwa, carry)
                    pltpu.async_copy(
                        table.at[ixs.at[pl.ds((2 * i + 2) * g, g)]], rwa, sma)
                    pltpu.make_async_copy(table.at[ixs.at[pl.ds(0, g)]], rwb,
                                          smb).wait()
                    carry = process((2 * i + 1) * g, rwb, carry)
                    return carry

                carry = lax.fori_loop(0, npi, pair, carry)
                pltpu.make_async_copy(table.at[ixs.at[pl.ds(0, g)]], rwa,
                                          sma).wait()
                return carry

            carry0 = (jnp.int32(cseg),) + tuple(neg for _ in range(nj))
            carry = lax.fori_loop(0, nsb, super_body, carry0)

            prev = carry[0]
            for j in range(nj):
                acc_v[pl.ds(prev * d + j * 16, 16)] = carry[1 + j]

            pltpu.sync_copy(acc_v.at[pl.ds(0, cseg * d)],
                            out.at[pl.ds(m0 * d, cseg * d)])
            return None

        lax.fori_loop(0, cpw, chunk_body, None)

    return k


def _norm_pallas(pooled):
    def body(pr, outr):
        v = pr[...]
        ss = jnp.sum(v * v, axis=0, keepdims=True)
        nrm = jnp.sqrt(ss)
        nrm = jnp.where(nrm == 0.0, 1.0, nrm)
        outr[...] = v / nrm

    return pl.pallas_call(
        body,
        out_shape=jax.ShapeDtypeStruct(pooled.shape, pooled.dtype),
    )(pooled)


def kernel(x, edge_index, cluster,
           w1_0, b1_0, g_0, beta_0, w2_0, b2_0,
           w1_1, b1_1, g_1, beta_1, w2_1, b2_1,
           w1_2, b1_2, g_2, beta_2, w2_2, b2_2):
    src = edge_index[0]
    dst = edge_index[1]

    dst_s, src_s = lax.sort_key_val(dst, src)
    src_p = jnp.concatenate(
        [src_s, jnp.zeros((EPAD - E,), jnp.int32)])
    dst_p = jnp.concatenate(
        [dst_s, jnp.full((EPAD - E,), NPAD, jnp.int32)])
    cb = jnp.searchsorted(
        dst_s, jnp.arange(0, NPAD + 1, 64, dtype=jnp.int32)).astype(jnp.int32)
    cb = jnp.concatenate([cb, jnp.full((15,), E, jnp.int32)])
    cb32 = jnp.searchsorted(
        dst_s, jnp.arange(0, NPAD + 1, 32, dtype=jnp.int32)).astype(jnp.int32)
    cb32 = jnp.concatenate([cb32, jnp.full((15,), E, jnp.int32)])

    rid_p = jnp.concatenate(
        [jnp.arange(N, dtype=jnp.int32), jnp.zeros((EPOOL - N,), jnp.int32)])
    clu_p = jnp.concatenate(
        [cluster, jnp.full((EPOOL - N,), P, jnp.int32)])
    cbp = jnp.searchsorted(
        cluster, jnp.arange(0, P + 1, 16, dtype=jnp.int32)).astype(jnp.int32)
    cbp = jnp.concatenate([cbp, jnp.full((15,), N, jnp.int32)])

    x_p = jnp.pad(x, ((0, NPAD - N), (0, 0)))

    segmax_128 = _make_segmax(128, 64, 5, NPAD, 176, 128, SB, EPAD)
    segmax_256 = _make_segmax(256, 64, 5, NPAD, 176, 128, SB, EPAD)
    segmax_512 = _make_segmax(512, 32, 10, NPAD, 336, 96, SB, EPAD)
    segmax_pool = _make_segmax(512, 16, 1, P, 48, 64, SBP, EPOOL)

    xt0 = _mlp_pallas([x_p], [w1_0], b1_0, g_0, beta_0, w2_0, b2_0)
    agg0 = segmax_128(xt0, src_p, dst_p, cb).reshape(NPAD, 128)
    xt1 = _mlp_pallas([xt0, agg0], [w1_1[:128], w1_1[128:]],
                      b1_1, g_1, beta_1, w2_1, b2_1)
    agg1 = segmax_256(xt1, src_p, dst_p, cb).reshape(NPAD, 256)
    xt2 = _mlp_pallas([xt1, agg1], [w1_2[:256], w1_2[256:]],
                      b1_2, g_2, beta_2, w2_2, b2_2)
    agg2 = segmax_512(xt2, src_p, dst_p, cb32).reshape(NPAD, 512)

    pool_a = segmax_pool(xt2, rid_p, clu_p, cbp).reshape(P, 512)
    pool_b = segmax_pool(agg2, rid_p, clu_p, cbp).reshape(P, 512)
    pooled = jnp.concatenate([pool_a, pool_b], axis=1)

    return _norm_pallas(pooled)
